# baseline (speedup 1.0000x reference)
import functools

import jax
import jax.numpy as jnp
from jax import lax
from jax.experimental import pallas as pl
from jax.experimental.pallas import tpu as pltpu
from jax.experimental.pallas import tpu_sc as plsc

N = 10000
NA = N + 8
D = 128
OUT = 128
NE = 320000

NUM_CORES = 2
NUM_SUBCORES = 16
LANES = 16

IROWS = 5120
NGRP = IROWS // 16
KMAX = NGRP // NUM_SUBCORES
K2MAX = NGRP // (2 * NUM_SUBCORES)
NPAIR = KMAX * 8
NCH2 = K2MAX * 16
ROWS_PER_TILE = 624


def _sc_segment_sums(src0, src1, dstm, x):
  mesh = plsc.VectorSubcoreMesh(
      core_axis_name="c", subcore_axis_name="s",
      num_cores=NUM_CORES, num_subcores=NUM_SUBCORES)

  @functools.partial(
      pl.kernel,
      out_type=[
          jax.ShapeDtypeStruct((N, D), jnp.float32),
          jax.ShapeDtypeStruct((N, D), jnp.float32),
          jax.ShapeDtypeStruct((N, D), jnp.float32),
          jax.ShapeDtypeStruct((N, D), jnp.float32),
      ],
      mesh=mesh,
      scratch_types=[
          pltpu.VMEM_SHARED((NA, D), jnp.float32),
          pltpu.VMEM((32, 64), jnp.int32),
          pltpu.VMEM((32, 64), jnp.int32),
          pltpu.VMEM((64, D), jnp.float32),
          pltpu.VMEM((64, D), jnp.float32),
          pltpu.SemaphoreType.DMA,
          pltpu.SemaphoreType.DMA,
          pltpu.SemaphoreType.DMA,
          pltpu.SemaphoreType.DMA,
          pltpu.SemaphoreType.DMA,
      ],
  )
  def k(src0_h, src1_h, dst_h, x_h, s0_out, s1_out, ca_out, cb_out,
        acc_sp, sidx_v, didx_v, rows0_v, rows1_v,
        gsem0, gsem1, ssem0, ssem1, isem):
    c = lax.axis_index("c")
    s = lax.axis_index("s")
    base = s * ROWS_PER_TILE

    zero16 = jnp.zeros((LANES,), jnp.float32)
    one16 = jnp.full((LANES,), 1.0, jnp.float32)

    def fill(ref, vec):
      def body(i, _):
        for j in range(D // LANES):
          ref[i, pl.ds(j * LANES, LANES)] = vec
        return 0
      lax.fori_loop(0, 64, body, 0)

    def zero_my_slice():
      for t in range(13):
        pltpu.sync_copy(rows0_v.at[pl.ds(0, 48)],
                        acc_sp.at[pl.ds(base + t * 48, 48)])

      @pl.when(s == NUM_SUBCORES - 1)
      def _():
        pltpu.sync_copy(rows0_v.at[pl.ds(0, 24)], acc_sp.at[pl.ds(N - 16, 24)])

    def write_my_slice(out_h):
      pltpu.sync_copy(acc_sp.at[pl.ds(base, ROWS_PER_TILE)],
                      out_h.at[pl.ds(base, ROWS_PER_TILE)])

      @pl.when(s == NUM_SUBCORES - 1)
      def _():
        pltpu.sync_copy(acc_sp.at[pl.ds(N - 16, 16)],
                        out_h.at[pl.ds(N - 16, 16)])

    def load_src_group(g, slot16):
      @pl.when(c == 0)
      def _():
        pltpu.async_copy(src0_h.at[pl.ds(g * 16, 16)],
                         sidx_v.at[pl.ds(slot16, 16)], isem)

      @pl.when(c == 1)
      def _():
        pltpu.async_copy(src1_h.at[pl.ds(g * 16, 16)],
                         sidx_v.at[pl.ds(slot16, 16)], isem)

    def wait_isem(n):
      for _ in range(n):
        pltpu.make_async_copy(dst_h.at[pl.ds(0, 16)],
                              didx_v.at[pl.ds(0, 16)], isem).wait()

    fill(rows0_v, zero16)
    fill(rows1_v, zero16)
    zero_my_slice()
    plsc.subcore_barrier()

    load_src_group(s, 0)
    pltpu.async_copy(dst_h.at[pl.ds(s * 16, 16)], didx_v.at[pl.ds(0, 16)],
                     isem)
    wait_isem(2)

    def pair(p, _):
      kk = p // 8
      slot16 = (kk % 2) * 16
      jj0 = slot16 + 2 * (p % 8)

      @pl.when((p % 8 == 0) & (kk > 0))
      def _():
        wait_isem(2)

      @pl.when(p > 0)
      def _():
        pltpu.make_async_copy(rows0_v, acc_sp.at[didx_v.at[jj0]],
                              ssem0).wait()
      g0a = pltpu.async_copy(x_h.at[sidx_v.at[jj0, pl.ds(0, 32)]],
                             rows0_v.at[pl.ds(0, 32)], gsem0)
      g0b = pltpu.async_copy(x_h.at[sidx_v.at[jj0, pl.ds(32, 32)]],
                             rows0_v.at[pl.ds(32, 32)], gsem0)

      @pl.when(p > 0)
      def _():
        pltpu.make_async_copy(rows1_v, acc_sp.at[didx_v.at[jj0 + 1]],
                              ssem1).wait()
      g1a = pltpu.async_copy(x_h.at[sidx_v.at[jj0 + 1, pl.ds(0, 32)]],
                             rows1_v.at[pl.ds(0, 32)], gsem1)
      g1b = pltpu.async_copy(x_h.at[sidx_v.at[jj0 + 1, pl.ds(32, 32)]],
                             rows1_v.at[pl.ds(32, 32)], gsem1)

      @pl.when((p % 8 == 0) & (kk < KMAX - 1))
      def _():
        g_next = s + NUM_SUBCORES * (kk + 1)
        nslot16 = ((kk + 1) % 2) * 16
        load_src_group(g_next, nslot16)
        pltpu.async_copy(dst_h.at[pl.ds(g_next * 16, 16)],
                         didx_v.at[pl.ds(nslot16, 16)], isem)

      g0a.wait()
      g0b.wait()
      pltpu.async_copy(rows0_v, acc_sp.at[didx_v.at[jj0]], ssem0, add=True)
      g1a.wait()
      g1b.wait()
      pltpu.async_copy(rows1_v, acc_sp.at[didx_v.at[jj0 + 1]], ssem1,
                       add=True)
      return 0
    lax.fori_loop(0, NPAIR, pair, 0)

    pltpu.make_async_copy(rows0_v, acc_sp.at[didx_v.at[0]], ssem0).wait()
    pltpu.make_async_copy(rows1_v, acc_sp.at[didx_v.at[1]], ssem1).wait()

    plsc.subcore_barrier()

    @pl.when(c == 0)
    def _():
      write_my_slice(s0_out)

    @pl.when(c == 1)
    def _():
      write_my_slice(s1_out)

    fill(rows0_v, zero16)
    zero_my_slice()
    fill(rows0_v, one16)
    plsc.subcore_barrier()

    gbase = c * (NGRP // 2) + s
    pltpu.async_copy(dst_h.at[pl.ds(gbase * 16, 16)], didx_v.at[pl.ds(0, 16)],
                     isem)
    wait_isem(1)

    def body2(p2, _):
      kk = p2 // 16
      slot16 = (kk % 2) * 16
      jj = slot16 + (p2 % 16)

      @pl.when((p2 % 16 == 0) & (kk > 0))
      def _():
        wait_isem(1)

      @pl.when((p2 % 16 == 0) & (kk < K2MAX - 1))
      def _():
        g_next = gbase + NUM_SUBCORES * (kk + 1)
        nslot16 = ((kk + 1) % 2) * 16
        pltpu.async_copy(dst_h.at[pl.ds(g_next * 16, 16)],
                         didx_v.at[pl.ds(nslot16, 16)], isem)

      @pl.when(p2 >= 2)
      def _():
        pltpu.make_async_copy(rows0_v, acc_sp.at[didx_v.at[jj]],
                              ssem0).wait()
      pltpu.async_copy(rows0_v, acc_sp.at[didx_v.at[jj]], ssem0, add=True)
      return 0
    lax.fori_loop(0, NCH2, body2, 0)

    pltpu.make_async_copy(rows0_v, acc_sp.at[didx_v.at[0]], ssem0).wait()
    pltpu.make_async_copy(rows0_v, acc_sp.at[didx_v.at[0]], ssem0).wait()

    plsc.subcore_barrier()

    @pl.when(c == 0)
    def _():
      write_my_slice(ca_out)

    @pl.when(c == 1)
    def _():
      write_my_slice(cb_out)

  return k(src0, src1, dstm, x)


def _dense_body(x_ref, s0_ref, s1_ref, ca_ref, cb_ref, cwt_ref, a0_ref,
                a1_ref, b_ref, o_ref):
  cnt = ca_ref[:, 0:1] + cb_ref[:, 0:1]
  recip = jnp.where(cnt > 0.0, 1.0 / cnt, 0.0)
  acc = jnp.dot(x_ref[:], cwt_ref[:], preferred_element_type=jnp.float32)
  acc = acc + jnp.dot(s0_ref[:] * recip, a0_ref[:],
                      preferred_element_type=jnp.float32)
  acc = acc + jnp.dot(s1_ref[:] * recip, a1_ref[:],
                      preferred_element_type=jnp.float32)
  o_ref[:] = acc + b_ref[:]


def _dense(x, s0, s1, cnt_a, cnt_b, cwt, a0, a1, cb):
  blk = 1000
  grid = (N // blk,)
  return pl.pallas_call(
      _dense_body,
      grid=grid,
      in_specs=[
          pl.BlockSpec((blk, D), lambda i: (i, 0)),
          pl.BlockSpec((blk, D), lambda i: (i, 0)),
          pl.BlockSpec((blk, D), lambda i: (i, 0)),
          pl.BlockSpec((blk, D), lambda i: (i, 0)),
          pl.BlockSpec((blk, D), lambda i: (i, 0)),
          pl.BlockSpec((D, OUT), lambda i: (0, 0)),
          pl.BlockSpec((D, OUT), lambda i: (0, 0)),
          pl.BlockSpec((D, OUT), lambda i: (0, 0)),
          pl.BlockSpec((1, OUT), lambda i: (0, 0)),
      ],
      out_specs=pl.BlockSpec((blk, OUT), lambda i: (i, 0)),
      out_shape=jax.ShapeDtypeStruct((N, OUT), jnp.float32),
  )(x, s0, s1, cnt_a, cnt_b, cwt, a0, a1, cb)


@jax.jit
def kernel(x, edge_index, A_rel, C_w, C_b):
  ei = edge_index.astype(jnp.int32)
  pad = IROWS * 64 - NE
  src0 = jnp.concatenate(
      [ei[0, 0::2], jnp.zeros((pad,), jnp.int32)]).reshape(IROWS, 64)
  src1 = jnp.concatenate(
      [ei[0, 1::2], jnp.zeros((pad,), jnp.int32)]).reshape(IROWS, 64)
  dstm = jnp.concatenate(
      [ei[1, 0::2],
       jnp.full((pad,), N, jnp.int32)]).reshape(IROWS, 64)

  s0, s1, cnt_a, cnt_b = _sc_segment_sums(src0, src1, dstm, x)

  cwt = C_w.T
  a0 = A_rel[:D, :]
  a1 = A_rel[D:, :]
  cb = C_b.reshape(1, OUT)
  return _dense(x, s0, s1, cnt_a, cnt_b, cwt, a0, a1, cb)


# speedup vs baseline: 1.2357x; 1.2357x over previous
import functools

import jax
import jax.numpy as jnp
from jax import lax
from jax.experimental import pallas as pl
from jax.experimental.pallas import tpu as pltpu
from jax.experimental.pallas import tpu_sc as plsc

N = 10000
NA = N + 8
D = 128
OUT = 128
NE = 320000

NUM_CORES = 2
NUM_SUBCORES = 16
LANES = 16

IROWS = 5120
NGRP = IROWS // 16
KMAX = NGRP // NUM_SUBCORES
K2MAX = NGRP // (2 * NUM_SUBCORES)
NPAIR = KMAX * 8
NCH2 = K2MAX * 16
ROWS_PER_TILE = 624


def _sc_segment_sums(eidx, x):
  mesh = plsc.VectorSubcoreMesh(
      core_axis_name="c", subcore_axis_name="s",
      num_cores=NUM_CORES, num_subcores=NUM_SUBCORES)

  @functools.partial(
      pl.kernel,
      out_type=[
          jax.ShapeDtypeStruct((N, D), jnp.float32),
          jax.ShapeDtypeStruct((N, D), jnp.float32),
          jax.ShapeDtypeStruct((N, D), jnp.float32),
          jax.ShapeDtypeStruct((N, D), jnp.float32),
      ],
      mesh=mesh,
      scratch_types=[
          pltpu.VMEM_SHARED((NA, D), jnp.float32),
          pltpu.VMEM((32, 64), jnp.int32),
          pltpu.VMEM((32, 64), jnp.int32),
          pltpu.VMEM((2048,), jnp.int32),
          pltpu.VMEM((2048,), jnp.int32),
          pltpu.VMEM((64, D), jnp.float32),
          pltpu.VMEM((64, D), jnp.float32),
          pltpu.SemaphoreType.DMA,
          pltpu.SemaphoreType.DMA,
          pltpu.SemaphoreType.DMA,
          pltpu.SemaphoreType.DMA,
          pltpu.SemaphoreType.DMA,
      ],
  )
  def k(eidx_h, x_h, s0_out, s1_out, ca_out, cb_out,
        acc_sp, sidx_v, didx_v, rawS_v, rawD_v, rows0_v, rows1_v,
        gsem0, gsem1, ssem0, ssem1, isem):
    c = lax.axis_index("c")
    s = lax.axis_index("s")
    base = s * ROWS_PER_TILE

    zero16 = jnp.zeros((LANES,), jnp.float32)
    one16 = jnp.full((LANES,), 1.0, jnp.float32)

    def fill(ref, vec):
      def body(i, _):
        for j in range(D // LANES):
          ref[i, pl.ds(j * LANES, LANES)] = vec
        return 0
      lax.fori_loop(0, 64, body, 0)

    def zero_my_slice():
      for t in range(13):
        pltpu.sync_copy(rows0_v.at[pl.ds(0, 48)],
                        acc_sp.at[pl.ds(base + t * 48, 48)])

      @pl.when(s == NUM_SUBCORES - 1)
      def _():
        pltpu.sync_copy(rows0_v.at[pl.ds(0, 24)], acc_sp.at[pl.ds(N - 16, 24)])

    def write_my_slice(out_h):
      pltpu.sync_copy(acc_sp.at[pl.ds(base, ROWS_PER_TILE)],
                      out_h.at[pl.ds(base, ROWS_PER_TILE)])

      @pl.when(s == NUM_SUBCORES - 1)
      def _():
        pltpu.sync_copy(acc_sp.at[pl.ds(N - 16, 16)],
                        out_h.at[pl.ds(N - 16, 16)])

    def load_raw_group(g, with_src):
      if with_src:
        pltpu.async_copy(eidx_h.at[0, pl.ds(g * 2048, 2048)], rawS_v, isem)
      pltpu.async_copy(eidx_h.at[1, pl.ds(g * 2048, 2048)], rawD_v, isem)

    def wait_isem(n):
      for _ in range(n):
        pltpu.make_async_copy(eidx_h.at[1, pl.ds(0, 2048)], rawD_v,
                              isem).wait()

    def extract_group(slot16, with_src):
      lane = lax.iota(jnp.int32, LANES)
      perm = jnp.where(lane < 8, 2 * lane, 2 * lane - LANES)

      dnums = lax.GatherDimensionNumbers(
          offset_dims=(), collapsed_slice_dims=(0,), start_index_map=(0,))

      def vgather(v, idx):
        return lax.gather(v, idx[:, None], dnums, (1,),
                          mode=lax.GatherScatterMode.PROMISE_IN_BOUNDS)

      def deint(ref, off, u):
        va = ref[pl.ds(u * 32, LANES)]
        vb = ref[pl.ds(u * 32 + LANES, LANES)]
        pidx = perm + off
        return jnp.where(lane < 8, vgather(va, pidx), vgather(vb, pidx))

      def eblk(u, _):
        r = u // 4
        tt = u % 4
        didx_v[slot16 + r, pl.ds(tt * LANES, LANES)] = deint(rawD_v, 0, u)
        if with_src:
          sidx_v[slot16 + r, pl.ds(tt * LANES, LANES)] = deint(rawS_v, c, u)
        return 0
      lax.fori_loop(0, 64, eblk, 0)

    fill(rows0_v, zero16)
    fill(rows1_v, zero16)
    zero_my_slice()
    plsc.subcore_barrier()

    load_raw_group(s, True)

    def pair(p, _):
      kk = p // 8
      slot16 = (kk % 2) * 16
      jj0 = slot16 + 2 * (p % 8)

      @pl.when(p % 8 == 0)
      def _():
        wait_isem(2)
        extract_group(slot16, True)

        @pl.when(kk < KMAX - 1)
        def _():
          load_raw_group(s + NUM_SUBCORES * (kk + 1), True)

      @pl.when(p > 0)
      def _():
        pltpu.make_async_copy(rows0_v, acc_sp.at[didx_v.at[jj0]],
                              ssem0).wait()
      g0a = pltpu.async_copy(x_h.at[sidx_v.at[jj0, pl.ds(0, 32)]],
                             rows0_v.at[pl.ds(0, 32)], gsem0)
      g0b = pltpu.async_copy(x_h.at[sidx_v.at[jj0, pl.ds(32, 32)]],
                             rows0_v.at[pl.ds(32, 32)], gsem0)

      @pl.when(p > 0)
      def _():
        pltpu.make_async_copy(rows1_v, acc_sp.at[didx_v.at[jj0 + 1]],
                              ssem1).wait()
      g1a = pltpu.async_copy(x_h.at[sidx_v.at[jj0 + 1, pl.ds(0, 32)]],
                             rows1_v.at[pl.ds(0, 32)], gsem1)
      g1b = pltpu.async_copy(x_h.at[sidx_v.at[jj0 + 1, pl.ds(32, 32)]],
                             rows1_v.at[pl.ds(32, 32)], gsem1)

      g0a.wait()
      g0b.wait()
      pltpu.async_copy(rows0_v, acc_sp.at[didx_v.at[jj0]], ssem0, add=True)
      g1a.wait()
      g1b.wait()
      pltpu.async_copy(rows1_v, acc_sp.at[didx_v.at[jj0 + 1]], ssem1,
                       add=True)
      return 0
    lax.fori_loop(0, NPAIR, pair, 0)

    pltpu.make_async_copy(rows0_v, acc_sp.at[didx_v.at[0]], ssem0).wait()
    pltpu.make_async_copy(rows1_v, acc_sp.at[didx_v.at[1]], ssem1).wait()

    plsc.subcore_barrier()

    @pl.when(c == 0)
    def _():
      write_my_slice(s0_out)

    @pl.when(c == 1)
    def _():
      write_my_slice(s1_out)

    fill(rows0_v, zero16)
    zero_my_slice()
    fill(rows0_v, one16)
    plsc.subcore_barrier()

    gbase = c * (NGRP // 2) + s
    load_raw_group(gbase, False)

    def body2(p2, _):
      kk = p2 // 16
      slot16 = (kk % 2) * 16
      jj = slot16 + (p2 % 16)

      @pl.when(p2 % 16 == 0)
      def _():
        wait_isem(1)
        extract_group(slot16, False)

        @pl.when(kk < K2MAX - 1)
        def _():
          load_raw_group(gbase + NUM_SUBCORES * (kk + 1), False)

      @pl.when(p2 >= 2)
      def _():
        pltpu.make_async_copy(rows0_v, acc_sp.at[didx_v.at[jj]],
                              ssem0).wait()
      pltpu.async_copy(rows0_v, acc_sp.at[didx_v.at[jj]], ssem0, add=True)
      return 0
    lax.fori_loop(0, NCH2, body2, 0)

    pltpu.make_async_copy(rows0_v, acc_sp.at[didx_v.at[0]], ssem0).wait()
    pltpu.make_async_copy(rows0_v, acc_sp.at[didx_v.at[0]], ssem0).wait()

    plsc.subcore_barrier()

    @pl.when(c == 0)
    def _():
      write_my_slice(ca_out)

    @pl.when(c == 1)
    def _():
      write_my_slice(cb_out)

  return k(eidx, x)


def _dense_body(x_ref, s0_ref, s1_ref, ca_ref, cb_ref, cwt_ref, a0_ref,
                a1_ref, b_ref, o_ref):
  cnt = ca_ref[:, 0:1] + cb_ref[:, 0:1]
  recip = jnp.where(cnt > 0.0, 1.0 / cnt, 0.0)
  acc = jnp.dot(x_ref[:], cwt_ref[:], preferred_element_type=jnp.float32)
  acc = acc + jnp.dot(s0_ref[:] * recip, a0_ref[:],
                      preferred_element_type=jnp.float32)
  acc = acc + jnp.dot(s1_ref[:] * recip, a1_ref[:],
                      preferred_element_type=jnp.float32)
  o_ref[:] = acc + b_ref[:]


def _dense(x, s0, s1, cnt_a, cnt_b, cwt, a0, a1, cb):
  blk = 1000
  grid = (N // blk,)
  return pl.pallas_call(
      _dense_body,
      grid=grid,
      in_specs=[
          pl.BlockSpec((blk, D), lambda i: (i, 0)),
          pl.BlockSpec((blk, D), lambda i: (i, 0)),
          pl.BlockSpec((blk, D), lambda i: (i, 0)),
          pl.BlockSpec((blk, D), lambda i: (i, 0)),
          pl.BlockSpec((blk, D), lambda i: (i, 0)),
          pl.BlockSpec((D, OUT), lambda i: (0, 0)),
          pl.BlockSpec((D, OUT), lambda i: (0, 0)),
          pl.BlockSpec((D, OUT), lambda i: (0, 0)),
          pl.BlockSpec((1, OUT), lambda i: (0, 0)),
      ],
      out_specs=pl.BlockSpec((blk, OUT), lambda i: (i, 0)),
      out_shape=jax.ShapeDtypeStruct((N, OUT), jnp.float32),
  )(x, s0, s1, cnt_a, cnt_b, cwt, a0, a1, cb)


@jax.jit
def kernel(x, edge_index, A_rel, C_w, C_b):
  ei = edge_index.astype(jnp.int32)
  pad = IROWS * 128 - 2 * NE
  pad_block = jnp.stack([jnp.zeros((pad,), jnp.int32),
                         jnp.full((pad,), N, jnp.int32)])
  eidx = jnp.concatenate([ei, pad_block], axis=1)

  s0, s1, cnt_a, cnt_b = _sc_segment_sums(eidx, x)

  cwt = C_w.T
  a0 = A_rel[:D, :]
  a1 = A_rel[D:, :]
  cb = C_b.reshape(1, OUT)
  return _dense(x, s0, s1, cnt_a, cnt_b, cwt, a0, a1, cb)
